# baseline (speedup 1.0000x reference)
import jax
import jax.numpy as jnp
from jax import lax
from jax.experimental import pallas as pl
from jax.experimental.pallas import tpu as pltpu
import jax.experimental.pallas.tpu_sc as plsc

N = 10000
E = 320000
D = 128

NC = 2
NS = 16
NW = NC * NS
KB = 80
NB = 126
EP = NW * NB * KB
EPW = NB * KB
NP = 10240
RPT = NP // NS
ZR = 160
DH = D // 2


def _spmm_body(xa_hbm, xb_hbm, col_hbm, row_hbm, w_hbm, outa_hbm, outb_hbm,
               col_v, row_v, w_v, rows_a, rows_b, zbuf, acc_sh, gsem, ssem):
    c = lax.axis_index("c")
    s = lax.axis_index("s")
    wid = c * NS + s

    pltpu.sync_copy(col_hbm.at[wid], col_v)
    pltpu.sync_copy(row_hbm.at[wid], row_v)
    pltpu.sync_copy(w_hbm.at[wid], w_v)

    def zrow(r, _):
        for j in range(DH // 16):
            zbuf[r, pl.ds(j * 16, 16)] = jnp.zeros((16,), jnp.float32)
        return _
    lax.fori_loop(0, ZR, zrow, None)

    def scale(buf, b):
        def group(g, _):
            wv = w_v[b, pl.ds(g * 16, 16)]
            for i in range(16):
                wsp = jnp.full((16,), wv[i], jnp.float32)
                e = g * 16 + i
                for j in range(DH // 16):
                    sl = pl.ds(j * 16, 16)
                    buf[e, sl] = buf[e, sl] * wsp
            return _
        lax.fori_loop(0, KB // 16, group, None)

    def one_pass(x_hbm, out_hbm):
        for k in range(RPT // ZR):
            pltpu.sync_copy(zbuf, acc_sh.at[pl.ds(s * RPT + k * ZR, ZR)])
        plsc.subcore_barrier()

        def wait_g(buf, b):
            pltpu.make_async_copy(x_hbm.at[col_v.at[b]], buf, gsem).wait()

        def wait_s(buf, b):
            pltpu.make_async_copy(buf, acc_sh.at[row_v.at[b]], ssem).wait()

        pltpu.async_copy(x_hbm.at[col_v.at[0]], rows_a, gsem)

        def pair(pb, _):
            b0 = 2 * pb
            b1 = b0 + 1
            wait_g(rows_a, b0)

            pltpu.async_copy(x_hbm.at[col_v.at[b1]], rows_b, gsem)
            scale(rows_a, b0)

            wait_g(rows_b, b1)

            @pl.when(b1 < NB - 1)
            def _():
                pltpu.async_copy(x_hbm.at[col_v.at[b1 + 1]], rows_a, gsem)
            scale(rows_b, b1)
            return _
        lax.fori_loop(0, NB // 2, pair, None)


        plsc.subcore_barrier()
        pltpu.sync_copy(acc_sh.at[pl.ds(s * RPT, RPT)],
                        out_hbm.at[c, pl.ds(s * RPT, RPT)])

    one_pass(xa_hbm, outa_hbm)
    one_pass(xb_hbm, outb_hbm)


_spmm = pl.kernel(
    _spmm_body,
    out_type=(jax.ShapeDtypeStruct((NC, NP, DH), jnp.float32),
              jax.ShapeDtypeStruct((NC, NP, DH), jnp.float32)),
    mesh=plsc.VectorSubcoreMesh(core_axis_name="c", subcore_axis_name="s"),
    compiler_params=pltpu.CompilerParams(use_tc_tiling_on_sc=False),
    scratch_types=[
        pltpu.VMEM((NB, KB), jnp.int32),
        pltpu.VMEM((NB, KB), jnp.int32),
        pltpu.VMEM((NB, KB), jnp.float32),
        pltpu.VMEM((KB, DH), jnp.float32),
        pltpu.VMEM((KB, DH), jnp.float32),
        pltpu.VMEM((ZR, DH), jnp.float32),
        pltpu.VMEM_SHARED((NP, DH), jnp.float32),
        pltpu.SemaphoreType.DMA,
        pltpu.SemaphoreType.DMA,
    ],
)

_BLK = 1000


def _dense_body(pa_ref, pb_ref, w_ref, o_ref):
    z = jnp.concatenate([pa_ref[0] + pa_ref[1], pb_ref[0] + pb_ref[1]],
                        axis=1)
    h = lax.dot_general(z, w_ref[...], (((1,), (1,)), ((), ())),
                        preferred_element_type=jnp.float32)
    o_ref[...] = jnp.maximum(h, 0.0)


_dense = pl.pallas_call(
    _dense_body,
    grid=(N // _BLK,),
    in_specs=[
        pl.BlockSpec((NC, _BLK, DH), lambda i: (0, i, 0)),
        pl.BlockSpec((NC, _BLK, DH), lambda i: (0, i, 0)),
        pl.BlockSpec((D, D), lambda i: (0, 0)),
    ],
    out_specs=pl.BlockSpec((_BLK, D), lambda i: (i, 0)),
    out_shape=jax.ShapeDtypeStruct((N, D), jnp.float32),
)


def _clf_body(x_ref, wc_ref, bc_ref, o_ref):
    logits = lax.dot_general(x_ref[...], wc_ref[...], (((1,), (1,)), ((), ())),
                             preferred_element_type=jnp.float32)
    o_ref[...] = logits + bc_ref[0, 0]


_clf = pl.pallas_call(
    _clf_body,
    grid=(N // _BLK,),
    in_specs=[
        pl.BlockSpec((_BLK, D), lambda i: (i, 0)),
        pl.BlockSpec((8, D), lambda i: (0, 0)),
        pl.BlockSpec((1, 1), lambda i: (0, 0)),
    ],
    out_specs=pl.BlockSpec((_BLK, 8), lambda i: (i, 0)),
    out_shape=jax.ShapeDtypeStruct((N, 8), jnp.float32),
)


@jax.jit
def kernel(features, adj_edge_index, adj_edge_weight, W1, W2, Wc, bc):
    pad = EP - E
    col = jnp.concatenate([adj_edge_index[1],
                           jnp.zeros((pad,), jnp.int32)]).reshape(NW, NB, KB)
    row = jnp.concatenate([adj_edge_index[0],
                           jnp.full((pad,), N, jnp.int32)]).reshape(NW, NB, KB)
    w = jnp.concatenate([adj_edge_weight,
                         jnp.zeros((pad,), jnp.float32)]).reshape(NW, NB, KB)
    x = features.astype(jnp.float32)
    Ws = jnp.stack([W1, W2])

    n_layers = lax.optimization_barrier(jnp.int32(2))

    def layer(i, xc):
        W = lax.dynamic_index_in_dim(Ws, i, keepdims=False)
        pa, pb = _spmm(xc[:, :DH], xc[:, DH:], col, row, w)
        return _dense(pa, pb, W)

    x2 = lax.fori_loop(0, n_layers, layer, x)

    Wc8 = jnp.zeros((8, D), jnp.float32).at[0].set(Wc[0])
    logits8 = _clf(x2, Wc8, bc.reshape(1, 1))
    return logits8[:, :1]


# speedup vs baseline: 1.2534x; 1.2534x over previous
import jax
import jax.numpy as jnp
from jax import lax
from jax.experimental import pallas as pl
from jax.experimental.pallas import tpu as pltpu
import jax.experimental.pallas.tpu_sc as plsc

N = 10000
E = 320000
D = 128

NC = 2
NS = 16
NW = NC * NS
KB = 80
NB = 126
EP = NW * NB * KB
EPW = NB * KB
NP = 10240
RPT = NP // NS
ZR = 160
DH = D // 2


def _spmm_body(xa_hbm, xb_hbm, col_hbm, row_hbm, w_hbm, outa_hbm, outb_hbm,
               col_v, row_v, w_v, rows_a, rows_b, zbuf, acc_sh, gsem, ssem):
    c = lax.axis_index("c")
    s = lax.axis_index("s")
    wid = c * NS + s

    pltpu.sync_copy(col_hbm.at[wid], col_v)
    pltpu.sync_copy(row_hbm.at[wid], row_v)
    pltpu.sync_copy(w_hbm.at[wid], w_v)

    def zrow(r, _):
        for j in range(DH // 16):
            zbuf[r, pl.ds(j * 16, 16)] = jnp.zeros((16,), jnp.float32)
        return _
    lax.fori_loop(0, ZR, zrow, None)

    def scale(buf, b):
        for g in range(KB // 16):
            wv = w_v[b, pl.ds(g * 16, 16)]
            for i in range(16):
                wsp = jnp.full((16,), wv[i], jnp.float32)
                e = g * 16 + i
                for j in range(DH // 16):
                    sl = pl.ds(j * 16, 16)
                    buf[e, sl] = buf[e, sl] * wsp

    def one_pass(x_hbm, out_hbm):
        for k in range(RPT // ZR):
            pltpu.sync_copy(zbuf, acc_sh.at[pl.ds(s * RPT + k * ZR, ZR)])
        plsc.subcore_barrier()

        def wait_g(buf, b):
            pltpu.make_async_copy(x_hbm.at[col_v.at[b]], buf, gsem).wait()

        def wait_s(buf, b):
            pltpu.make_async_copy(buf, acc_sh.at[row_v.at[b]], ssem).wait()

        pltpu.async_copy(x_hbm.at[col_v.at[0]], rows_a, gsem)

        def pair(pb, _):
            b0 = 2 * pb
            b1 = b0 + 1
            wait_g(rows_a, b0)

            @pl.when(b0 > 0)
            def _():
                wait_s(rows_b, b0 - 1)
            pltpu.async_copy(x_hbm.at[col_v.at[b1]], rows_b, gsem)
            scale(rows_a, b0)
            pltpu.async_copy(rows_a, acc_sh.at[row_v.at[b0]], ssem, add=True)

            wait_g(rows_b, b1)

            @pl.when(b1 < NB - 1)
            def _():
                wait_s(rows_a, b0)
                pltpu.async_copy(x_hbm.at[col_v.at[b1 + 1]], rows_a, gsem)
            scale(rows_b, b1)
            pltpu.async_copy(rows_b, acc_sh.at[row_v.at[b1]], ssem, add=True)
            return _
        lax.fori_loop(0, NB // 2, pair, None)

        wait_s(rows_a, NB - 2)
        wait_s(rows_b, NB - 1)

        plsc.subcore_barrier()
        pltpu.sync_copy(acc_sh.at[pl.ds(s * RPT, RPT)],
                        out_hbm.at[c, pl.ds(s * RPT, RPT)])

    one_pass(xa_hbm, outa_hbm)
    one_pass(xb_hbm, outb_hbm)


_spmm = pl.kernel(
    _spmm_body,
    out_type=(jax.ShapeDtypeStruct((NC, NP, DH), jnp.float32),
              jax.ShapeDtypeStruct((NC, NP, DH), jnp.float32)),
    mesh=plsc.VectorSubcoreMesh(core_axis_name="c", subcore_axis_name="s"),
    compiler_params=pltpu.CompilerParams(use_tc_tiling_on_sc=False),
    scratch_types=[
        pltpu.VMEM((NB, KB), jnp.int32),
        pltpu.VMEM((NB, KB), jnp.int32),
        pltpu.VMEM((NB, KB), jnp.float32),
        pltpu.VMEM((KB, DH), jnp.float32),
        pltpu.VMEM((KB, DH), jnp.float32),
        pltpu.VMEM((ZR, DH), jnp.float32),
        pltpu.VMEM_SHARED((NP, DH), jnp.float32),
        pltpu.SemaphoreType.DMA,
        pltpu.SemaphoreType.DMA,
    ],
)

_BLK = 1000


def _dense_body(pa_ref, pb_ref, w_ref, o_ref):
    z = jnp.concatenate([pa_ref[0] + pa_ref[1], pb_ref[0] + pb_ref[1]],
                        axis=1)
    h = lax.dot_general(z, w_ref[...], (((1,), (1,)), ((), ())),
                        preferred_element_type=jnp.float32)
    o_ref[...] = jnp.maximum(h, 0.0)


_dense = pl.pallas_call(
    _dense_body,
    grid=(N // _BLK,),
    in_specs=[
        pl.BlockSpec((NC, _BLK, DH), lambda i: (0, i, 0)),
        pl.BlockSpec((NC, _BLK, DH), lambda i: (0, i, 0)),
        pl.BlockSpec((D, D), lambda i: (0, 0)),
    ],
    out_specs=pl.BlockSpec((_BLK, D), lambda i: (i, 0)),
    out_shape=jax.ShapeDtypeStruct((N, D), jnp.float32),
)


def _clf_body(x_ref, wc_ref, bc_ref, o_ref):
    logits = lax.dot_general(x_ref[...], wc_ref[...], (((1,), (1,)), ((), ())),
                             preferred_element_type=jnp.float32)
    o_ref[...] = logits + bc_ref[0, 0]


_clf = pl.pallas_call(
    _clf_body,
    grid=(N // _BLK,),
    in_specs=[
        pl.BlockSpec((_BLK, D), lambda i: (i, 0)),
        pl.BlockSpec((8, D), lambda i: (0, 0)),
        pl.BlockSpec((1, 1), lambda i: (0, 0)),
    ],
    out_specs=pl.BlockSpec((_BLK, 8), lambda i: (i, 0)),
    out_shape=jax.ShapeDtypeStruct((N, 8), jnp.float32),
)


@jax.jit
def kernel(features, adj_edge_index, adj_edge_weight, W1, W2, Wc, bc):
    pad = EP - E
    col = jnp.concatenate([adj_edge_index[1],
                           jnp.zeros((pad,), jnp.int32)]).reshape(NW, NB, KB)
    row = jnp.concatenate([adj_edge_index[0],
                           jnp.full((pad,), N, jnp.int32)]).reshape(NW, NB, KB)
    w = jnp.concatenate([adj_edge_weight,
                         jnp.zeros((pad,), jnp.float32)]).reshape(NW, NB, KB)
    x = features.astype(jnp.float32)
    Ws = jnp.stack([W1, W2])

    n_layers = lax.optimization_barrier(jnp.int32(2))

    def layer(i, xc):
        W = lax.dynamic_index_in_dim(Ws, i, keepdims=False)
        pa, pb = _spmm(xc[:, :DH], xc[:, DH:], col, row, w)
        return _dense(pa, pb, W)

    x2 = lax.fori_loop(0, n_layers, layer, x)

    Wc8 = jnp.zeros((8, D), jnp.float32).at[0].set(Wc[0])
    logits8 = _clf(x2, Wc8, bc.reshape(1, 1))
    return logits8[:, :1]


# speedup vs baseline: 1.5933x; 1.2712x over previous
import jax
import jax.numpy as jnp
from jax import lax
from jax.experimental import pallas as pl
from jax.experimental.pallas import tpu as pltpu
import jax.experimental.pallas.tpu_sc as plsc

N = 10000
E = 320000
D = 128

NC = 2
NS = 16
NW = NC * NS
KB = 80
NB = 126
EP = NW * NB * KB
EPW = NB * KB
NP = 10240
RPT = NP // NS
ZR = 160
DH = D // 2


def _spmm_body(xa_hbm, xb_hbm, col_hbm, row_hbm, w_hbm, outa_hbm, outb_hbm,
               col_v, row_v, w_v, rows_a, rows_b, rows_c, zbuf, acc_sh,
               gsem, ssem):
    c = lax.axis_index("c")
    s = lax.axis_index("s")
    wid = c * NS + s

    pltpu.sync_copy(col_hbm.at[wid], col_v)
    pltpu.sync_copy(row_hbm.at[wid], row_v)
    pltpu.sync_copy(w_hbm.at[wid], w_v)

    def zrow(r, _):
        for j in range(DH // 16):
            zbuf[r, pl.ds(j * 16, 16)] = jnp.zeros((16,), jnp.float32)
        return _
    lax.fori_loop(0, ZR, zrow, None)

    def scale(buf, b):
        for g in range(KB // 16):
            wv = w_v[b, pl.ds(g * 16, 16)]
            for i in range(16):
                wsp = jnp.full((16,), wv[i], jnp.float32)
                e = g * 16 + i
                for j in range(DH // 16):
                    sl = pl.ds(j * 16, 16)
                    buf[e, sl] = buf[e, sl] * wsp

    def one_pass(x_hbm, out_hbm):
        for k in range(RPT // ZR):
            pltpu.sync_copy(zbuf, acc_sh.at[pl.ds(s * RPT + k * ZR, ZR)])
        plsc.subcore_barrier()

        def wait_g(buf, b):
            pltpu.make_async_copy(x_hbm.at[col_v.at[b]], buf, gsem).wait()

        def wait_s(buf, b):
            pltpu.make_async_copy(buf, acc_sh.at[row_v.at[b]], ssem).wait()

        bufs = (rows_a, rows_b, rows_c)
        pltpu.async_copy(x_hbm.at[col_v.at[0]], rows_a, gsem)
        pltpu.async_copy(x_hbm.at[col_v.at[1]], rows_b, gsem)

        def triple(pt, _):
            for k in range(3):
                b = 3 * pt + k
                cur = bufs[k]
                nxt = bufs[(k + 2) % 3]
                wait_g(cur, b)

                @pl.when(b < NB - 2)
                def _():
                    @pl.when(b > 0)
                    def _():
                        wait_s(nxt, b - 1)
                    pltpu.async_copy(x_hbm.at[col_v.at[b + 2]], nxt, gsem)
                scale(cur, b)
                pltpu.async_copy(cur, acc_sh.at[row_v.at[b]], ssem, add=True)
            return _
        lax.fori_loop(0, NB // 3, triple, None)

        wait_s(rows_a, NB - 3)
        wait_s(rows_b, NB - 2)
        wait_s(rows_c, NB - 1)

        plsc.subcore_barrier()
        pltpu.sync_copy(acc_sh.at[pl.ds(s * RPT, RPT)],
                        out_hbm.at[c, pl.ds(s * RPT, RPT)])

    one_pass(xa_hbm, outa_hbm)
    one_pass(xb_hbm, outb_hbm)


_spmm = pl.kernel(
    _spmm_body,
    out_type=(jax.ShapeDtypeStruct((NC, NP, DH), jnp.float32),
              jax.ShapeDtypeStruct((NC, NP, DH), jnp.float32)),
    mesh=plsc.VectorSubcoreMesh(core_axis_name="c", subcore_axis_name="s"),
    compiler_params=pltpu.CompilerParams(use_tc_tiling_on_sc=False),
    scratch_types=[
        pltpu.VMEM((NB, KB), jnp.int32),
        pltpu.VMEM((NB, KB), jnp.int32),
        pltpu.VMEM((NB, KB), jnp.float32),
        pltpu.VMEM((KB, DH), jnp.float32),
        pltpu.VMEM((KB, DH), jnp.float32),
        pltpu.VMEM((KB, DH), jnp.float32),
        pltpu.VMEM((ZR, DH), jnp.float32),
        pltpu.VMEM_SHARED((NP, DH), jnp.float32),
        pltpu.SemaphoreType.DMA,
        pltpu.SemaphoreType.DMA,
    ],
)

_BLK = 1000


def _dense_body(pa_ref, pb_ref, w_ref, o_ref):
    z = jnp.concatenate([pa_ref[0] + pa_ref[1], pb_ref[0] + pb_ref[1]],
                        axis=1)
    h = lax.dot_general(z, w_ref[...], (((1,), (1,)), ((), ())),
                        preferred_element_type=jnp.float32)
    o_ref[...] = jnp.maximum(h, 0.0)


_dense = pl.pallas_call(
    _dense_body,
    grid=(N // _BLK,),
    in_specs=[
        pl.BlockSpec((NC, _BLK, DH), lambda i: (0, i, 0)),
        pl.BlockSpec((NC, _BLK, DH), lambda i: (0, i, 0)),
        pl.BlockSpec((D, D), lambda i: (0, 0)),
    ],
    out_specs=pl.BlockSpec((_BLK, D), lambda i: (i, 0)),
    out_shape=jax.ShapeDtypeStruct((N, D), jnp.float32),
)


def _clf_body(x_ref, wc_ref, bc_ref, o_ref):
    logits = lax.dot_general(x_ref[...], wc_ref[...], (((1,), (1,)), ((), ())),
                             preferred_element_type=jnp.float32)
    o_ref[...] = logits + bc_ref[0, 0]


_clf = pl.pallas_call(
    _clf_body,
    grid=(N // _BLK,),
    in_specs=[
        pl.BlockSpec((_BLK, D), lambda i: (i, 0)),
        pl.BlockSpec((8, D), lambda i: (0, 0)),
        pl.BlockSpec((1, 1), lambda i: (0, 0)),
    ],
    out_specs=pl.BlockSpec((_BLK, 8), lambda i: (i, 0)),
    out_shape=jax.ShapeDtypeStruct((N, 8), jnp.float32),
)


@jax.jit
def kernel(features, adj_edge_index, adj_edge_weight, W1, W2, Wc, bc):
    pad = EP - E
    col = jnp.concatenate([adj_edge_index[1],
                           jnp.zeros((pad,), jnp.int32)]).reshape(NW, NB, KB)
    row = jnp.concatenate([adj_edge_index[0],
                           jnp.full((pad,), N, jnp.int32)]).reshape(NW, NB, KB)
    w = jnp.concatenate([adj_edge_weight,
                         jnp.zeros((pad,), jnp.float32)]).reshape(NW, NB, KB)
    x = features.astype(jnp.float32)
    Ws = jnp.stack([W1, W2])

    n_layers = lax.optimization_barrier(jnp.int32(2))

    def layer(i, xc):
        W = lax.dynamic_index_in_dim(Ws, i, keepdims=False)
        pa, pb = _spmm(xc[:, :DH], xc[:, DH:], col, row, w)
        return _dense(pa, pb, W)

    x2 = lax.fori_loop(0, n_layers, layer, x)

    Wc8 = jnp.zeros((8, D), jnp.float32).at[0].set(Wc[0])
    logits8 = _clf(x2, Wc8, bc.reshape(1, 1))
    return logits8[:, :1]
